# Initial kernel scaffold; baseline (speedup 1.0000x reference)
import jax
import jax.numpy as jnp
import numpy as np
from jax import lax
from jax.experimental import pallas as pl
from jax.experimental.pallas import tpu as pltpu

B, N, MAX_NODE = 4, 2048, 407
T_STEPS = 10
NP = 512
CH = 256
NSEG = B * MAX_NODE
CHUNK = 512
NEG = -3.0e38
NEG_TEST = -1.0e30


def _mega_kernel(seg_ref, x_ref, nodes_ref, ain_ref, aout_ref,
                 w1_ref, w2_ref, w3_ref, win_ref, wout_ref,
                 wz1_ref, wz2_ref, wz3_ref,
                 wr1_ref, wr2_ref, wr3_ref,
                 wh1_ref, wh2_ref, wh3_ref,
                 wos_ref, woa_ref, wg_ref,
                 out_ref,
                 hbuf_ref, acc_ref, ann_ref, st_ref):
    f32 = jnp.float32
    w1 = w1_ref[...]
    w2 = w2_ref[...]
    w3 = w3_ref[...]

    acc_ref[...] = jnp.full((NSEG, CH), NEG, f32)

    for c in range(N * B // CHUNK):
        xs = x_ref[pl.ds(c * CHUNK, CHUNK), :]
        h = jnp.maximum(xs @ w1, 0.0)
        h = jnp.maximum(h @ w2, 0.0)
        hbuf_ref[...] = h @ w3

        def scat_body(p, _):
            s = seg_ref[c * CHUNK + p]
            row = hbuf_ref[pl.ds(p, 1), :]
            cur = acc_ref[pl.ds(s, 1), :]
            acc_ref[pl.ds(s, 1), :] = jnp.maximum(cur, row)
            return 0

        lax.fori_loop(0, CHUNK, scat_body, 0)

    for b in range(B):
        blk = jnp.zeros((NP, 512), f32)
        a = acc_ref[pl.ds(b * MAX_NODE, MAX_NODE), :]
        fl = jnp.where(a > NEG_TEST, a, 0.0)
        blk = lax.dynamic_update_slice(blk, fl, (0, 0))
        blk = lax.dynamic_update_slice(blk, nodes_ref[b], (0, CH))
        ann_ref[b] = blk
        st_ref[b] = blk

    win = win_ref[...]
    wout = wout_ref[...]
    wz1, wz2, wz3 = wz1_ref[...], wz2_ref[...], wz3_ref[...]
    wr1, wr2, wr3 = wr1_ref[...], wr2_ref[...], wr3_ref[...]
    wh1, wh2, wh3 = wh1_ref[...], wh2_ref[...], wh3_ref[...]

    def step(t, _):
        for b in range(B):
            s = st_ref[b]
            ai = ain_ref[b] @ (s @ win)
            ao = aout_ref[b] @ (s @ wout)
            z = jax.nn.sigmoid(ai @ wz1 + ao @ wz2 + s @ wz3)
            r = jax.nn.sigmoid(ai @ wr1 + ao @ wr2 + s @ wr3)
            hc = jnp.tanh(ai @ wh1 + ao @ wh2 + (r * s) @ wh3)
            st_ref[b] = (1.0 - z) * s + z * hc
        return 0

    lax.fori_loop(0, T_STEPS, step, 0)

    wos = wos_ref[...]
    woa = woa_ref[...]
    wg = wg_ref[...]
    row_id = lax.broadcasted_iota(jnp.int32, (NP, 1024), 0)
    for b in range(B):
        s = st_ref[b]
        a = ann_ref[b]
        fn = jnp.tanh(s @ wos + a @ woa)
        fgm = fn @ wg
        fgm = jnp.where(row_id < MAX_NODE, fgm, NEG)
        fg = jnp.max(fgm, axis=0)
        out_ref[b, pl.ds(0, 1024), :] = jnp.broadcast_to(
            fg[:, None], (1024, MAX_NODE))
        fnT = fn.T
        out_ref[b, pl.ds(1024, 512), :] = fnT[:, :MAX_NODE]
        aT = a[:, :CH].T
        out_ref[b, pl.ds(1536, 256), :] = aT[:, :MAX_NODE]


def kernel(xyz13, overseg_idx, nodes, graph, W1, W2, W3, Win, Wout,
           Wz, Wr, Wh, Wo, Wg):
    f32 = jnp.float32
    x = xyz13[:, :, :12].reshape(B * N, 12)
    seg = (jnp.arange(B, dtype=jnp.int32)[:, None] * MAX_NODE
           + overseg_idx.astype(jnp.int32)).reshape(-1)
    a_in = jnp.pad(graph[:, :, :MAX_NODE],
                   ((0, 0), (0, NP - MAX_NODE), (0, NP - MAX_NODE)))
    a_out = jnp.pad(graph[:, :, MAX_NODE:],
                    ((0, 0), (0, NP - MAX_NODE), (0, NP - MAX_NODE)))
    wz1, wz2, wz3 = Wz[:512], Wz[512:1024], Wz[1024:]
    wr1, wr2, wr3 = Wr[:512], Wr[512:1024], Wr[1024:]
    wh1, wh2, wh3 = Wh[:512], Wh[512:1024], Wh[1024:]
    wos = Wo[:512]
    woa = jnp.pad(Wo[512:], ((0, 512 - 262), (0, 0)))

    vmem = pl.BlockSpec(memory_space=pltpu.VMEM)
    smem = pl.BlockSpec(memory_space=pltpu.SMEM)
    out = pl.pallas_call(
        _mega_kernel,
        out_shape=jax.ShapeDtypeStruct((B, 1792, MAX_NODE), f32),
        in_specs=[smem] + [vmem] * 21,
        out_specs=vmem,
        scratch_shapes=[
            pltpu.VMEM((CHUNK, CH), f32),
            pltpu.VMEM((NSEG, CH), f32),
            pltpu.VMEM((B, NP, 512), f32),
            pltpu.VMEM((B, NP, 512), f32),
        ],
    )(seg, x, nodes, a_in, a_out, W1, W2, W3, Win, Wout,
      wz1, wz2, wz3, wr1, wr2, wr3, wh1, wh2, wh3, wos, woa, Wg)
    return out


# speedup vs baseline: 1.4268x; 1.4268x over previous
import jax
import jax.numpy as jnp
import numpy as np
from jax import lax
from jax.experimental import pallas as pl
from jax.experimental.pallas import tpu as pltpu

B, N, MAX_NODE = 4, 2048, 407
T_STEPS = 10
NP = 512
CH = 256
NSEG = B * MAX_NODE
CHUNK = 512
NEG = -3.0e38
NEG_TEST = -1.0e30


def _mlp_segmax_kernel(seg_ref, x_ref, w1_ref, w2_ref, w3_ref,
                       acc_ref, hbuf_ref):
    w1 = w1_ref[...]
    w2 = w2_ref[...]
    w3 = w3_ref[...]
    acc_ref[...] = jnp.full((B * NP, CH), NEG, jnp.float32)

    for c in range(N * B // CHUNK):
        xs = x_ref[pl.ds(c * CHUNK, CHUNK), :]
        h = jnp.maximum(xs @ w1, 0.0)
        h = jnp.maximum(h @ w2, 0.0)
        hbuf_ref[...] = h @ w3

        def scat_body(p, _):
            s = seg_ref[c * CHUNK + p]
            row = hbuf_ref[pl.ds(p, 1), :]
            cur = acc_ref[pl.ds(s, 1), :]
            acc_ref[pl.ds(s, 1), :] = jnp.maximum(cur, row)
            return 0

        lax.fori_loop(0, CHUNK, scat_body, 0)


def _ggnn_kernel(acc_ref, nodes_ref, ain_ref, aout_ref,
                 win_ref, wout_ref,
                 wz1_ref, wz2_ref, wz3_ref,
                 wr1_ref, wr2_ref, wr3_ref,
                 wh1_ref, wh2_ref, wh3_ref,
                 wos_ref, woa_ref, wg_ref,
                 out_ref,
                 ann_ref, st_ref):
    f32 = jnp.float32
    ann_ref[...] = jnp.zeros((NP, 512), f32)
    b = pl.program_id(0)
    a = acc_ref[pl.ds(pl.multiple_of(b * NP, NP), MAX_NODE), :]
    fl = jnp.where(a > NEG_TEST, a, 0.0)
    ann_ref[pl.ds(0, MAX_NODE), pl.ds(0, CH)] = fl
    ann_ref[pl.ds(0, MAX_NODE), pl.ds(CH, 6)] = nodes_ref[0]
    st_ref[...] = ann_ref[...]

    win = win_ref[...]
    wout = wout_ref[...]
    wz1, wz2, wz3 = wz1_ref[...], wz2_ref[...], wz3_ref[...]
    wr1, wr2, wr3 = wr1_ref[...], wr2_ref[...], wr3_ref[...]
    wh1, wh2, wh3 = wh1_ref[...], wh2_ref[...], wh3_ref[...]
    a_in = ain_ref[0]
    a_out = aout_ref[0]

    def step(t, _):
        s = st_ref[...]
        ai = a_in @ (s @ win)
        ao = a_out @ (s @ wout)
        z = jax.nn.sigmoid(ai @ wz1 + ao @ wz2 + s @ wz3)
        r = jax.nn.sigmoid(ai @ wr1 + ao @ wr2 + s @ wr3)
        hc = jnp.tanh(ai @ wh1 + ao @ wh2 + (r * s) @ wh3)
        st_ref[...] = (1.0 - z) * s + z * hc
        return 0

    lax.fori_loop(0, T_STEPS, step, 0)

    wos = wos_ref[...]
    woa = woa_ref[...]
    wg = wg_ref[...]
    row_id = lax.broadcasted_iota(jnp.int32, (NP, 1024), 0)
    s = st_ref[...]
    a = ann_ref[...]
    fn = jnp.tanh(s @ wos + a @ woa)
    fgm = fn @ wg
    fgm = jnp.where(row_id < MAX_NODE, fgm, NEG)
    fg = jnp.max(fgm, axis=0)
    out_ref[0, pl.ds(0, 1024), :] = jnp.broadcast_to(
        fg[:, None], (1024, MAX_NODE))
    fnT = fn.T
    out_ref[0, pl.ds(1024, 512), :] = fnT[:, :MAX_NODE]
    aT = a[:, :CH].T
    out_ref[0, pl.ds(1536, 256), :] = aT[:, :MAX_NODE]


def kernel(xyz13, overseg_idx, nodes, graph, W1, W2, W3, Win, Wout,
           Wz, Wr, Wh, Wo, Wg):
    f32 = jnp.float32
    x = xyz13[:, :, :12].reshape(B * N, 12)
    seg = (jnp.arange(B, dtype=jnp.int32)[:, None] * NP
           + overseg_idx.astype(jnp.int32)).reshape(-1)
    a_in = jnp.pad(graph[:, :, :MAX_NODE],
                   ((0, 0), (0, NP - MAX_NODE), (0, NP - MAX_NODE)))
    a_out = jnp.pad(graph[:, :, MAX_NODE:],
                    ((0, 0), (0, NP - MAX_NODE), (0, NP - MAX_NODE)))
    wz1, wz2, wz3 = Wz[:512], Wz[512:1024], Wz[1024:]
    wr1, wr2, wr3 = Wr[:512], Wr[512:1024], Wr[1024:]
    wh1, wh2, wh3 = Wh[:512], Wh[512:1024], Wh[1024:]
    wos = Wo[:512]
    woa = jnp.pad(Wo[512:], ((0, 512 - 262), (0, 0)))

    vmem = pl.BlockSpec(memory_space=pltpu.VMEM)
    smem = pl.BlockSpec(memory_space=pltpu.SMEM)

    acc = pl.pallas_call(
        _mlp_segmax_kernel,
        out_shape=jax.ShapeDtypeStruct((B * NP, CH), f32),
        in_specs=[smem] + [vmem] * 4,
        out_specs=vmem,
        scratch_shapes=[pltpu.VMEM((CHUNK, CH), f32)],
    )(seg, x, W1, W2, W3)

    wspec = pl.BlockSpec((512, 512), lambda b: (0, 0))
    out = pl.pallas_call(
        _ggnn_kernel,
        grid=(B,),
        out_shape=jax.ShapeDtypeStruct((B, 1792, MAX_NODE), f32),
        in_specs=[
            pl.BlockSpec(memory_space=pltpu.VMEM),
            pl.BlockSpec((1, MAX_NODE, 6), lambda b: (b, 0, 0)),
            pl.BlockSpec((1, NP, NP), lambda b: (b, 0, 0)),
            pl.BlockSpec((1, NP, NP), lambda b: (b, 0, 0)),
        ] + [wspec] * 13 + [pl.BlockSpec((512, 1024), lambda b: (0, 0))],
        out_specs=pl.BlockSpec((1, 1792, MAX_NODE), lambda b: (b, 0, 0)),
        scratch_shapes=[
            pltpu.VMEM((NP, 512), f32),
            pltpu.VMEM((NP, 512), f32),
        ],
    )(acc, nodes, a_in, a_out, Win, Wout,
      wz1, wz2, wz3, wr1, wr2, wr3, wh1, wh2, wh3, wos, woa, Wg)
    return out
